# baseline (speedup 1.0000x reference)
import jax
import jax.numpy as jnp
from jax import lax
from jax.experimental import pallas as pl
from jax.experimental.pallas import tpu as pltpu
from jax.experimental.pallas import tpu_sc as plsc

_VOCAB = 1000000
_BATCH = 4096
_SEQ = 200
_DIM = 64
_PDIM = 128
_LANES = 16
_NC = 2
_NS = 16
_NW = _NC * _NS
_BPW = _BATCH // _NW
_NCH = _DIM // _LANES
_NBUF = 2
_UNROLL = 4

_W = 128
_VMAIN = (_VOCAB // _W) * _W
_NBLK = _VMAIN // _W
_FMT_SLOTS = 246


def _fmt_body(tabT_hbm, tail_hbm, pad_hbm, in_v, st_v, sem_i, sem_o):
    wid = lax.axis_index("s") * _NC + lax.axis_index("c")
    nmine = (_NBLK - wid + _NW - 1) // _NW

    def start_in(s, b):
        blk = wid + _NW * s
        pltpu.async_copy(
            tabT_hbm.at[:, pl.ds(blk * _W, _W)], in_v.at[b], sem_i.at[b])

    def drain_in(s, b):
        blk = wid + _NW * s
        pltpu.make_async_copy(
            tabT_hbm.at[:, pl.ds(blk * _W, _W)], in_v.at[b], sem_i.at[b],
        ).wait()

    def start_out(s, b):
        blk = wid + _NW * s
        pltpu.async_copy(
            st_v.at[b], pad_hbm.at[pl.ds(blk * _W, _W)], sem_o.at[b])

    def drain_out(s, b):
        blk = wid + _NW * s
        pltpu.make_async_copy(
            st_v.at[b], pad_hbm.at[pl.ds(blk * _W, _W)], sem_o.at[b],
        ).wait()

    def transpose(b):
        rowidx = [jnp.arange(_LANES, dtype=jnp.int32) + _LANES * c
                  for c in range(_NCH)]

        @pl.loop(0, _W)
        def _col(v):
            colidx = jnp.broadcast_to(v, (_LANES,)).astype(jnp.int32)
            for c in range(_NCH):
                g = plsc.load_gather(in_v.at[b], [rowidx[c], colidx])
                st_v[b, v, pl.ds(_LANES * c, _LANES)] = g
                st_v[b, v, pl.ds(_DIM + _LANES * c, _LANES)] = g

    for b in range(2):
        @pl.when(b < nmine)
        def _prime():
            start_in(b, b)

    @pl.loop(0, _FMT_SLOTS, step=2)
    def _blk(i):
        for b in range(2):
            s = i + b

            @pl.when(s < nmine)
            def _work():
                drain_in(s, b)

                @pl.when(s >= 2)
                def _dr():
                    drain_out(s - 2, b)

                transpose(b)
                start_out(s, b)

                @pl.when(s + 2 < nmine)
                def _next():
                    start_in(s + 2, b)

    for b in range(2):
        last = nmine - 1 - ((nmine - 1 - b) % 2)

        @pl.when(last >= 0)
        def _fin():
            drain_out(last, b)

    @pl.when(wid == 0)
    def _tail():
        pltpu.sync_copy(tail_hbm, pad_hbm.at[pl.ds(_VMAIN, _VOCAB - _VMAIN)])


def _encode_body(idx_hbm, table_hbm, out_hbm, idx_v, rows_v, out_v, sems):
    wid = lax.axis_index("s") * _NC + lax.axis_index("c")
    base = wid * _BPW
    pltpu.sync_copy(idx_hbm.at[pl.ds(base * _SEQ, _BPW * _SEQ)], idx_v)

    def start(e, b):
        pltpu.async_copy(
            table_hbm.at[idx_v.at[pl.ds(e * _SEQ, _SEQ)]],
            rows_v.at[b],
            sems.at[b],
        )

    def drain(e, b):
        pltpu.make_async_copy(
            table_hbm.at[idx_v.at[pl.ds(e * _SEQ, _SEQ)]],
            rows_v.at[b],
            sems.at[b],
        ).wait()

    def accum(e, b):
        def body(i, acc):
            s = i * _UNROLL
            for u in range(_UNROLL):
                acc = tuple(
                    acc[c] + rows_v[b, s + u, pl.ds(_LANES * c, _LANES)]
                    for c in range(_NCH))
            return acc

        zero = jnp.zeros((_LANES,), jnp.float32)
        acc = lax.fori_loop(0, _SEQ // _UNROLL, body, (zero,) * _NCH,
                            unroll=False)
        scale = jnp.float32(1.0 / _SEQ)
        for c in range(_NCH):
            out_v[e, pl.ds(_LANES * c, _LANES)] = acc[c] * scale

    for b in range(_NBUF):
        start(b, b)

    @pl.loop(0, _BPW, step=_NBUF)
    def _elem(e):
        for b in range(_NBUF):
            ee = e + b
            drain(ee, b)
            accum(ee, b)

            @pl.when(ee + _NBUF < _BPW)
            def _prefetch():
                start(ee + _NBUF, b)

    pltpu.sync_copy(out_v, out_hbm.at[pl.ds(base, _BPW)])


def kernel(token_ids, table):
    idx_flat = token_ids.astype(jnp.int32).reshape(_BATCH * _SEQ)
    table_t = jnp.transpose(table)
    tail = jnp.pad(table[_VMAIN:], ((0, 0), (0, _PDIM - _DIM)))
    mesh = plsc.VectorSubcoreMesh(core_axis_name="c", subcore_axis_name="s")
    fmt_k = pl.kernel(
        _fmt_body,
        out_type=jax.ShapeDtypeStruct((_VOCAB, _PDIM), jnp.float32),
        mesh=mesh,
        compiler_params=pltpu.CompilerParams(use_tc_tiling_on_sc=True,
                                             needs_layout_passes=False),
        scratch_types=[
            pltpu.VMEM((2, _DIM, _W), jnp.float32),
            pltpu.VMEM((2, _W, _PDIM), jnp.float32),
            pltpu.SemaphoreType.DMA((2,)),
            pltpu.SemaphoreType.DMA((2,)),
        ],
    )
    table_pad = fmt_k(table_t, tail)
    enc_k = pl.kernel(
        _encode_body,
        out_type=jax.ShapeDtypeStruct((_BATCH, _DIM), jnp.float32),
        mesh=mesh,
        compiler_params=pltpu.CompilerParams(use_tc_tiling_on_sc=True),
        scratch_types=[
            pltpu.VMEM((_BPW * _SEQ,), jnp.int32),
            pltpu.VMEM((_NBUF, _SEQ, _PDIM), jnp.float32),
            pltpu.VMEM((_BPW, _DIM), jnp.float32),
            pltpu.SemaphoreType.DMA((_NBUF,)),
        ],
    )
    return enc_k(idx_flat, table_pad)


# speedup vs baseline: 1.1641x; 1.1641x over previous
import jax
import jax.numpy as jnp
from jax import lax
from jax.experimental import pallas as pl
from jax.experimental.pallas import tpu as pltpu
from jax.experimental.pallas import tpu_sc as plsc

_VOCAB = 1000000
_BATCH = 4096
_SEQ = 200
_DIM = 64
_PDIM = 128
_LANES = 16
_NC = 2
_NS = 16
_NW = _NC * _NS
_BPW = _BATCH // _NW
_NCH = _DIM // _LANES
_NBUF = 2
_UNROLL = 4

_W = 128
_VMAIN = (_VOCAB // _W) * _W
_NBLK = _VMAIN // _W
_FMT_SLOTS = 246


def _fmt_body(tabT_hbm, tail_hbm, pad_hbm, in_v, st_v, sem_i, sem_o):
    wid = lax.axis_index("s") * _NC + lax.axis_index("c")
    nmine = (_NBLK - wid + _NW - 1) // _NW

    def start_in(s, b):
        blk = wid + _NW * s
        pltpu.async_copy(
            tabT_hbm.at[:, pl.ds(blk * _W, _W)], in_v.at[b], sem_i.at[b])

    def drain_in(s, b):
        blk = wid + _NW * s
        pltpu.make_async_copy(
            tabT_hbm.at[:, pl.ds(blk * _W, _W)], in_v.at[b], sem_i.at[b],
        ).wait()

    def start_out(s, b):
        blk = wid + _NW * s
        pltpu.async_copy(
            st_v.at[b], pad_hbm.at[pl.ds(blk * _W, _W)], sem_o.at[b])

    def drain_out(s, b):
        blk = wid + _NW * s
        pltpu.make_async_copy(
            st_v.at[b], pad_hbm.at[pl.ds(blk * _W, _W)], sem_o.at[b],
        ).wait()

    def transpose(b):
        rowidx = [jnp.arange(_LANES, dtype=jnp.int32) + _LANES * c
                  for c in range(_NCH)]

        @pl.loop(0, _W, step=8)
        def _col(v):
            colidx = jnp.broadcast_to(v, (_LANES,)).astype(jnp.int32)
            for u in range(8):
                gs = [plsc.load_gather(in_v.at[b], [rowidx[c], colidx + u])
                      for c in range(_NCH)]
                for c in range(_NCH):
                    st_v[b, v + u, pl.ds(_LANES * c, _LANES)] = gs[c]
                    st_v[b, v + u, pl.ds(_DIM + _LANES * c, _LANES)] = gs[c]

    for b in range(2):
        @pl.when(b < nmine)
        def _prime():
            start_in(b, b)

    @pl.loop(0, _FMT_SLOTS, step=2)
    def _blk(i):
        for b in range(2):
            s = i + b

            @pl.when(s < nmine)
            def _work():
                drain_in(s, b)

                @pl.when(s >= 2)
                def _dr():
                    drain_out(s - 2, b)

                transpose(b)
                start_out(s, b)

                @pl.when(s + 2 < nmine)
                def _next():
                    start_in(s + 2, b)

    for b in range(2):
        last = nmine - 1 - ((nmine - 1 - b) % 2)

        @pl.when(last >= 0)
        def _fin():
            drain_out(last, b)

    @pl.when(wid == 0)
    def _tail():
        pltpu.sync_copy(tail_hbm, pad_hbm.at[pl.ds(_VMAIN, _VOCAB - _VMAIN)])


def _encode_body(idx_hbm, table_hbm, out_hbm, idx_v, rows_v, out_v, sems):
    wid = lax.axis_index("s") * _NC + lax.axis_index("c")
    base = wid * _BPW
    pltpu.sync_copy(idx_hbm.at[pl.ds(base * _SEQ, _BPW * _SEQ)], idx_v)

    def start(e, b):
        pltpu.async_copy(
            table_hbm.at[idx_v.at[pl.ds(e * _SEQ, _SEQ)]],
            rows_v.at[b],
            sems.at[b],
        )

    def drain(e, b):
        pltpu.make_async_copy(
            table_hbm.at[idx_v.at[pl.ds(e * _SEQ, _SEQ)]],
            rows_v.at[b],
            sems.at[b],
        ).wait()

    def accum(e, b):
        def body(i, acc):
            s = i * _UNROLL
            for u in range(_UNROLL):
                acc = tuple(
                    acc[c] + rows_v[b, s + u, pl.ds(_LANES * c, _LANES)]
                    for c in range(_NCH))
            return acc

        zero = jnp.zeros((_LANES,), jnp.float32)
        acc = lax.fori_loop(0, _SEQ // _UNROLL, body, (zero,) * _NCH,
                            unroll=False)
        scale = jnp.float32(1.0 / _SEQ)
        for c in range(_NCH):
            out_v[e, pl.ds(_LANES * c, _LANES)] = acc[c] * scale

    for b in range(_NBUF):
        start(b, b)

    @pl.loop(0, _BPW, step=_NBUF)
    def _elem(e):
        for b in range(_NBUF):
            ee = e + b
            drain(ee, b)
            accum(ee, b)

            @pl.when(ee + _NBUF < _BPW)
            def _prefetch():
                start(ee + _NBUF, b)

    pltpu.sync_copy(out_v, out_hbm.at[pl.ds(base, _BPW)])


def kernel(token_ids, table):
    idx_flat = token_ids.astype(jnp.int32).reshape(_BATCH * _SEQ)
    table_t = jnp.transpose(table)
    tail = jnp.pad(table[_VMAIN:], ((0, 0), (0, _PDIM - _DIM)))
    mesh = plsc.VectorSubcoreMesh(core_axis_name="c", subcore_axis_name="s")
    fmt_k = pl.kernel(
        _fmt_body,
        out_type=jax.ShapeDtypeStruct((_VOCAB, _PDIM), jnp.float32),
        mesh=mesh,
        compiler_params=pltpu.CompilerParams(use_tc_tiling_on_sc=True,
                                             needs_layout_passes=False),
        scratch_types=[
            pltpu.VMEM((2, _DIM, _W), jnp.float32),
            pltpu.VMEM((2, _W, _PDIM), jnp.float32),
            pltpu.SemaphoreType.DMA((2,)),
            pltpu.SemaphoreType.DMA((2,)),
        ],
    )
    table_pad = fmt_k(table_t, tail)
    enc_k = pl.kernel(
        _encode_body,
        out_type=jax.ShapeDtypeStruct((_BATCH, _DIM), jnp.float32),
        mesh=mesh,
        compiler_params=pltpu.CompilerParams(use_tc_tiling_on_sc=True),
        scratch_types=[
            pltpu.VMEM((_BPW * _SEQ,), jnp.int32),
            pltpu.VMEM((_NBUF, _SEQ, _PDIM), jnp.float32),
            pltpu.VMEM((_BPW, _DIM), jnp.float32),
            pltpu.SemaphoreType.DMA((_NBUF,)),
        ],
    )
    return enc_k(idx_flat, table_pad)
